# baseline (device time: 1952688 ns/iter reference)
import jax
import jax.numpy as jnp
from jax import lax
from jax.experimental import pallas as pl
from jax.experimental.pallas import tpu as pltpu

N_DEV = 32
N_RINGS = 4


def _hamiltonian_cycle():
    path = []
    for y in range(4):
        zs = range(4) if y % 2 == 0 else range(3, -1, -1)
        path.extend((y, z) for z in zs)
    cyc = [(0, y, z) for (y, z) in path]
    cyc += [(1, y, z) for (y, z) in reversed(path)]

    def midx(x, y, z):
        return z * 8 + y * 2 + (x if y % 2 == 0 else 1 - x)

    ring = [midx(*c) for c in cyc]
    assert sorted(ring) == list(range(N_DEV))
    inv = [0] * N_DEV
    for p, m in enumerate(ring):
        inv[m] = p
    return ring, inv

_RING, _INV = _hamiltonian_cycle()


def _ring_allreduce(y, scal):
    M, N = y.shape
    Q = M // N_RINGS
    CH = Q // N_DEV
    QCH = 64

    def body(scal_ref, y_ref, out_ref, send_bufs, recv_bufs, qbuf,
             send_sems, recv_sems, load_sems, store_sems,
             qload_sems, qstore_sems, credits_sem, amax_ref):
        pos = scal_ref[0]
        left = scal_ref[1]
        right = scal_ref[2]
        amax_ref[0] = jnp.float32(0.0)

        barrier_sem = pltpu.get_barrier_semaphore()
        for nbr in (left, right):
            pl.semaphore_signal(
                barrier_sem, inc=1,
                device_id=(nbr,), device_id_type=pl.DeviceIdType.MESH,
            )
        pl.semaphore_wait(barrier_sem, 2)

        def fwd(rr):
            return rr < 2

        def peer(rr):
            return right if fwd(rr) else left

        def crediter(rr):
            return left if fwd(rr) else right

        def row0(rr, idx):
            return rr * Q + idx * CH

        def load(rr, idx, slot):
            cp = pltpu.make_async_copy(
                y_ref.at[pl.ds(row0(rr, idx), CH), :],
                send_bufs.at[rr, slot], load_sems.at[rr, slot])
            cp.start()
            return cp

        def store(rr, src_slot, idx, from_send):
            buf = send_bufs if from_send else recv_bufs
            cp = pltpu.make_async_copy(
                buf.at[rr, src_slot],
                out_ref.at[pl.ds(row0(rr, idx), CH), :],
                store_sems.at[rr])
            cp.start()
            return cp

        def rdma(rr, slot):
            return pltpu.make_async_remote_copy(
                src_ref=send_bufs.at[rr, slot],
                dst_ref=recv_bufs.at[rr, slot],
                send_sem=send_sems.at[rr, slot],
                recv_sem=recv_sems.at[rr, slot],
                device_id=(peer(rr),),
                device_id_type=pl.DeviceIdType.MESH,
            )

        def give_credits():
            for rr in range(N_RINGS):
                pl.semaphore_signal(
                    credits_sem.at[rr], inc=1,
                    device_id=(crediter(rr),),
                    device_id_type=pl.DeviceIdType.MESH,
                )

        def wait_credits(g):
            @pl.when(g >= 2)
            def _():
                for rr in range(N_RINGS):
                    pl.semaphore_wait(credits_sem.at[rr], 1)

        def rs_acc_idx(rr, g):
            if fwd(rr):
                return lax.rem(pos - (g + 1) + N_DEV, N_DEV)
            return lax.rem(pos + (g + 1), N_DEV)

        def ag_idx(rr, t):
            if fwd(rr):
                return lax.rem(pos - t + N_DEV, N_DEV)
            return lax.rem(pos + t, N_DEV)

        for rr in range(N_RINGS):
            load(rr, pos, 0)
        for rr in range(N_RINGS):
            pltpu.make_async_copy(
                y_ref.at[pl.ds(row0(rr, pos), CH), :],
                send_bufs.at[rr, 0], load_sems.at[rr, 0]).wait()

        def rs_body(g, carry):
            slot = lax.rem(g, 2)
            nxt = lax.rem(g + 1, 2)
            wait_credits(g)
            rd = [rdma(rr, slot) for rr in range(N_RINGS)]
            for r_ in rd:
                r_.start()

            @pl.when(g >= 1)
            def _():
                for rr in range(N_RINGS):
                    rdma(rr, nxt).wait_send()

            for rr in range(N_RINGS):
                load(rr, rs_acc_idx(rr, g), nxt)
            for rr in range(N_RINGS):
                pltpu.make_async_copy(
                    y_ref.at[pl.ds(row0(rr, rs_acc_idx(rr, g)), CH), :],
                    send_bufs.at[rr, nxt], load_sems.at[rr, nxt]).wait()
                rd[rr].wait_recv()
                send_bufs[rr, nxt] = send_bufs[rr, nxt] + recv_bufs[rr, slot]
            give_credits()
            return carry

        lax.fori_loop(0, N_DEV - 1, rs_body, 0)

        amax = jnp.float32(0.0)
        sts = []
        for rr in range(N_RINGS):
            own = lax.rem(pos + (1 if fwd(rr) else N_DEV - 1), N_DEV)
            sts.append(store(rr, 1, own, from_send=True))
            amax = jnp.maximum(amax, jnp.max(jnp.abs(send_bufs[rr, 1])))
        amax_ref[0] = amax
        for rr, st in enumerate(sts):
            st.wait()

        def ag_body(t, carry):
            g = (N_DEV - 1) + t
            slot = lax.rem(g, 2)
            nxt = lax.rem(g + 1, 2)
            wait_credits(g)
            rd = [rdma(rr, slot) for rr in range(N_RINGS)]
            for r_ in rd:
                r_.start()
            for rr in range(N_RINGS):
                rdma(rr, nxt).wait_send()
            for rr in range(N_RINGS):
                rd[rr].wait_recv()
                store(rr, slot, ag_idx(rr, t), from_send=False)

                @pl.when(t < N_DEV - 2)
                def _():
                    send_bufs[rr, nxt] = recv_bufs[rr, slot]

            for rr in range(N_RINGS):
                pltpu.make_async_copy(
                    recv_bufs.at[rr, slot],
                    out_ref.at[pl.ds(row0(rr, ag_idx(rr, t)), CH), :],
                    store_sems.at[rr]).wait()
            give_credits()
            m = amax_ref[0]
            for rr in range(N_RINGS):
                m = jnp.maximum(m, jnp.max(jnp.abs(recv_bufs[rr, slot])))
            amax_ref[0] = m
            return carry

        lax.fori_loop(0, N_DEV - 1, ag_body, 0)

        for rr in range(N_RINGS):
            rdma(rr, 1).wait_send()
        for rr in range(N_RINGS):
            pl.semaphore_wait(credits_sem.at[rr], 2)

        amax = amax_ref[0]
        scale = amax / jnp.float32(448.0)
        inv_scale = jnp.float32(448.0) / amax

        def snap(v):
            a = jnp.abs(v) * inv_scale
            u = lax.bitcast_convert_type(a, jnp.int32)
            lsb = jnp.bitwise_and(lax.shift_right_logical(u, 20), 1)
            ur = jnp.bitwise_and(u + lsb + ((1 << 19) - 1),
                                 jnp.int32(~((1 << 20) - 1)))
            n_norm = lax.bitcast_convert_type(ur, jnp.float32)
            magic = jnp.float32(16384.0)
            n_sub = (a + magic) - magic
            snapped = jnp.where(a >= jnp.float32(2.0 ** -6), n_norm, n_sub)
            snapped = jnp.minimum(snapped, jnp.float32(448.0))
            return jnp.sign(v) * snapped * scale

        n_q = M // QCH

        def qload_cp(k):
            slot = lax.rem(k, 2)
            return pltpu.make_async_copy(
                out_ref.at[pl.ds(k * QCH, QCH), :], qbuf.at[slot],
                qload_sems.at[slot])

        def qstore_cp(k):
            slot = lax.rem(k, 2)
            return pltpu.make_async_copy(
                qbuf.at[slot], out_ref.at[pl.ds(k * QCH, QCH), :],
                qstore_sems.at[slot])

        qload_cp(0).start()

        def qbody(k, carry):
            slot = lax.rem(k, 2)

            @pl.when(k >= 1)
            def _():
                qstore_cp(k - 1).wait()

            @pl.when(k + 1 < n_q)
            def _():
                qload_cp(k + 1).start()

            qload_cp(k).wait()
            qbuf[slot] = snap(qbuf[slot])
            qstore_cp(k).start()
            return carry

        lax.fori_loop(0, n_q, qbody, 0)
        qstore_cp(n_q - 1).wait()

    return pl.pallas_call(
        body,
        out_shape=jax.ShapeDtypeStruct((M, N), jnp.float32),
        in_specs=[
            pl.BlockSpec(memory_space=pltpu.MemorySpace.SMEM),
            pl.BlockSpec(memory_space=pl.ANY),
        ],
        out_specs=pl.BlockSpec(memory_space=pl.ANY),
        scratch_shapes=[
            pltpu.VMEM((N_RINGS, 2, CH, N), jnp.float32),
            pltpu.VMEM((N_RINGS, 2, CH, N), jnp.float32),
            pltpu.VMEM((2, QCH, N), jnp.float32),
            pltpu.SemaphoreType.DMA((N_RINGS, 2)),
            pltpu.SemaphoreType.DMA((N_RINGS, 2)),
            pltpu.SemaphoreType.DMA((N_RINGS, 2)),
            pltpu.SemaphoreType.DMA((N_RINGS,)),
            pltpu.SemaphoreType.DMA((2,)),
            pltpu.SemaphoreType.DMA((2,)),
            pltpu.SemaphoreType.REGULAR((N_RINGS,)),
            pltpu.SMEM((1,), jnp.float32),
        ],
        compiler_params=pltpu.CompilerParams(collective_id=0),
    )(scal, y)


def kernel(x, w_mat):
    y = jnp.dot(x, w_mat, preferred_element_type=jnp.float32,
                precision=lax.Precision.HIGHEST)
    r = lax.axis_index("i")
    ring = jnp.asarray(_RING, jnp.int32)
    pos = jnp.asarray(_INV, jnp.int32)[r]
    right = ring[lax.rem(pos + 1, N_DEV)]
    left = ring[lax.rem(pos - 1 + N_DEV, N_DEV)]
    scal = jnp.stack([pos, left, right]).astype(jnp.int32)
    return _ring_allreduce(y, scal)


# device time: 1781016 ns/iter; 1.0964x vs baseline; 1.0964x over previous
import jax
import jax.numpy as jnp
from jax import lax
from jax.experimental import pallas as pl
from jax.experimental.pallas import tpu as pltpu

N_DEV = 32
N_RINGS = 4


def _hamiltonian_cycle():
    path = []
    for y in range(4):
        zs = range(4) if y % 2 == 0 else range(3, -1, -1)
        path.extend((y, z) for z in zs)
    cyc = [(0, y, z) for (y, z) in path]
    cyc += [(1, y, z) for (y, z) in reversed(path)]

    def midx(x, y, z):
        return z * 8 + y * 2 + (x if y % 2 == 0 else 1 - x)

    ring = [midx(*c) for c in cyc]
    assert sorted(ring) == list(range(N_DEV))
    inv = [0] * N_DEV
    for p, m in enumerate(ring):
        inv[m] = p
    return ring, inv

_RING, _INV = _hamiltonian_cycle()


def _ring_allreduce(y, scal):
    M, N = y.shape
    Q = M // N_RINGS
    CH = Q // N_DEV
    QCH = 64

    def body(scal_ref, y_ref, out_ref, send_bufs, recv_bufs, qbuf,
             send_sems, recv_sems, load_sems, store_sems,
             qload_sems, qstore_sems, credits_sem, amax_ref):
        pos = scal_ref[0]
        left = scal_ref[1]
        right = scal_ref[2]
        amax_ref[0] = jnp.float32(0.0)

        barrier_sem = pltpu.get_barrier_semaphore()
        for nbr in (left, right):
            pl.semaphore_signal(
                barrier_sem, inc=1,
                device_id=(nbr,), device_id_type=pl.DeviceIdType.MESH,
            )
        pl.semaphore_wait(barrier_sem, 2)

        def fwd(rr):
            return rr < 2

        def peer(rr):
            return right if fwd(rr) else left

        def crediter(rr):
            return left if fwd(rr) else right

        def row0(rr, idx):
            return rr * Q + idx * CH

        def load(rr, idx, slot):
            cp = pltpu.make_async_copy(
                y_ref.at[pl.ds(row0(rr, idx), CH), :],
                send_bufs.at[rr, slot], load_sems.at[rr, slot])
            cp.start()
            return cp

        def store(rr, src_slot, idx, from_send):
            buf = send_bufs if from_send else recv_bufs
            cp = pltpu.make_async_copy(
                buf.at[rr, src_slot],
                out_ref.at[pl.ds(row0(rr, idx), CH), :],
                store_sems.at[rr])
            cp.start()
            return cp

        def rdma(rr, slot):
            return pltpu.make_async_remote_copy(
                src_ref=send_bufs.at[rr, slot],
                dst_ref=recv_bufs.at[rr, slot],
                send_sem=send_sems.at[rr, slot],
                recv_sem=recv_sems.at[rr, slot],
                device_id=(peer(rr),),
                device_id_type=pl.DeviceIdType.MESH,
            )

        def give_credits():
            for rr in range(N_RINGS):
                pl.semaphore_signal(
                    credits_sem.at[rr], inc=1,
                    device_id=(crediter(rr),),
                    device_id_type=pl.DeviceIdType.MESH,
                )

        def wait_credits(g):
            @pl.when(g >= 2)
            def _():
                for rr in range(N_RINGS):
                    pl.semaphore_wait(credits_sem.at[rr], 1)

        def rs_acc_idx(rr, g):
            if fwd(rr):
                return lax.rem(pos - (g + 1) + N_DEV, N_DEV)
            return lax.rem(pos + (g + 1), N_DEV)

        def ag_idx(rr, t):
            if fwd(rr):
                return lax.rem(pos - t + N_DEV, N_DEV)
            return lax.rem(pos + t, N_DEV)

        def load_wait(rr, idx, slot):
            pltpu.make_async_copy(
                y_ref.at[pl.ds(row0(rr, idx), CH), :],
                send_bufs.at[rr, slot], load_sems.at[rr, slot]).wait()

        for rr in range(N_RINGS):
            load(rr, pos, 0)
        for rr in range(N_RINGS):
            load_wait(rr, pos, 0)
            rdma(rr, 0).start()
            load(rr, rs_acc_idx(rr, 0), 1)

        def rs_tick(t, carry):
            slot = lax.rem(t, 2)
            nxt = lax.rem(t + 1, 2)
            for rr in range(N_RINGS):
                load_wait(rr, rs_acc_idx(rr, t), nxt)
                rdma(rr, slot).wait_recv()
                send_bufs[rr, nxt] = (send_bufs[rr, nxt]
                                      + recv_bufs[rr, slot])
                pl.semaphore_signal(
                    credits_sem.at[rr], inc=1,
                    device_id=(crediter(rr),),
                    device_id_type=pl.DeviceIdType.MESH,
                )

                @pl.when(t + 1 < N_DEV - 1)
                def _():
                    @pl.when(t >= 1)
                    def _():
                        pl.semaphore_wait(credits_sem.at[rr], 1)
                    rdma(rr, nxt).start()

                rdma(rr, slot).wait_send()

                @pl.when(t + 1 < N_DEV - 1)
                def _():
                    load(rr, rs_acc_idx(rr, t + 1), slot)
            return carry

        lax.fori_loop(0, N_DEV - 1, rs_tick, 0)

        for rr in range(N_RINGS):
            pl.semaphore_wait(credits_sem.at[rr], 1)
            rdma(rr, 1).start()

        amax = jnp.float32(0.0)
        sts = []
        for rr in range(N_RINGS):
            own = lax.rem(pos + (1 if fwd(rr) else N_DEV - 1), N_DEV)
            sts.append(store(rr, 1, own, from_send=True))
            amax = jnp.maximum(amax, jnp.max(jnp.abs(send_bufs[rr, 1])))
        amax_ref[0] = amax
        for st in sts:
            st.wait()

        def ag_tick(t, carry):
            r_s = lax.rem(t + 1, 2)
            s_s = lax.rem(t, 2)
            for rr in range(N_RINGS):
                rdma(rr, r_s).wait_recv()
                store(rr, r_s, ag_idx(rr, t), from_send=False)

                @pl.when(t >= 1)
                def _():
                    rdma(rr, s_s).wait_send()

                @pl.when(t + 1 < N_DEV - 1)
                def _():
                    send_bufs[rr, s_s] = recv_bufs[rr, r_s]
                    pl.semaphore_wait(credits_sem.at[rr], 1)
                    rdma(rr, s_s).start()

                pltpu.make_async_copy(
                    recv_bufs.at[rr, r_s],
                    out_ref.at[pl.ds(row0(rr, ag_idx(rr, t)), CH), :],
                    store_sems.at[rr]).wait()
                pl.semaphore_signal(
                    credits_sem.at[rr], inc=1,
                    device_id=(crediter(rr),),
                    device_id_type=pl.DeviceIdType.MESH,
                )
            m = amax_ref[0]
            for rr in range(N_RINGS):
                m = jnp.maximum(m, jnp.max(jnp.abs(recv_bufs[rr, r_s])))
            amax_ref[0] = m
            return carry

        lax.fori_loop(0, N_DEV - 1, ag_tick, 0)

        for rr in range(N_RINGS):
            rdma(rr, 1).wait_send()
        for rr in range(N_RINGS):
            pl.semaphore_wait(credits_sem.at[rr], 2)

        amax = amax_ref[0]
        scale = amax / jnp.float32(448.0)
        inv_scale = jnp.float32(448.0) / amax

        def snap(v):
            a = jnp.abs(v) * inv_scale
            u = lax.bitcast_convert_type(a, jnp.int32)
            lsb = jnp.bitwise_and(lax.shift_right_logical(u, 20), 1)
            ur = jnp.bitwise_and(u + lsb + ((1 << 19) - 1),
                                 jnp.int32(~((1 << 20) - 1)))
            n_norm = lax.bitcast_convert_type(ur, jnp.float32)
            magic = jnp.float32(16384.0)
            n_sub = (a + magic) - magic
            snapped = jnp.where(a >= jnp.float32(2.0 ** -6), n_norm, n_sub)
            snapped = jnp.minimum(snapped, jnp.float32(448.0))
            return jnp.sign(v) * snapped * scale

        n_q = M // QCH

        def qload_cp(k):
            slot = lax.rem(k, 2)
            return pltpu.make_async_copy(
                out_ref.at[pl.ds(k * QCH, QCH), :], qbuf.at[slot],
                qload_sems.at[slot])

        def qstore_cp(k):
            slot = lax.rem(k, 2)
            return pltpu.make_async_copy(
                qbuf.at[slot], out_ref.at[pl.ds(k * QCH, QCH), :],
                qstore_sems.at[slot])

        qload_cp(0).start()

        def qbody(k, carry):
            slot = lax.rem(k, 2)

            @pl.when(k >= 1)
            def _():
                qstore_cp(k - 1).wait()

            @pl.when(k + 1 < n_q)
            def _():
                qload_cp(k + 1).start()

            qload_cp(k).wait()
            qbuf[slot] = snap(qbuf[slot])
            qstore_cp(k).start()
            return carry

        lax.fori_loop(0, n_q, qbody, 0)
        qstore_cp(n_q - 1).wait()

    return pl.pallas_call(
        body,
        out_shape=jax.ShapeDtypeStruct((M, N), jnp.float32),
        in_specs=[
            pl.BlockSpec(memory_space=pltpu.MemorySpace.SMEM),
            pl.BlockSpec(memory_space=pl.ANY),
        ],
        out_specs=pl.BlockSpec(memory_space=pl.ANY),
        scratch_shapes=[
            pltpu.VMEM((N_RINGS, 2, CH, N), jnp.float32),
            pltpu.VMEM((N_RINGS, 2, CH, N), jnp.float32),
            pltpu.VMEM((2, QCH, N), jnp.float32),
            pltpu.SemaphoreType.DMA((N_RINGS, 2)),
            pltpu.SemaphoreType.DMA((N_RINGS, 2)),
            pltpu.SemaphoreType.DMA((N_RINGS, 2)),
            pltpu.SemaphoreType.DMA((N_RINGS,)),
            pltpu.SemaphoreType.DMA((2,)),
            pltpu.SemaphoreType.DMA((2,)),
            pltpu.SemaphoreType.REGULAR((N_RINGS,)),
            pltpu.SMEM((1,), jnp.float32),
        ],
        compiler_params=pltpu.CompilerParams(collective_id=0),
    )(scal, y)


def kernel(x, w_mat):
    y = jnp.dot(x, w_mat, preferred_element_type=jnp.float32,
                precision=lax.Precision.HIGHEST)
    r = lax.axis_index("i")
    ring = jnp.asarray(_RING, jnp.int32)
    pos = jnp.asarray(_INV, jnp.int32)[r]
    right = ring[lax.rem(pos + 1, N_DEV)]
    left = ring[lax.rem(pos - 1 + N_DEV, N_DEV)]
    scal = jnp.stack([pos, left, right]).astype(jnp.int32)
    return _ring_allreduce(y, scal)


# device time: 1672539 ns/iter; 1.1675x vs baseline; 1.0649x over previous
import jax
import jax.numpy as jnp
from jax import lax
from jax.experimental import pallas as pl
from jax.experimental.pallas import tpu as pltpu

N_DEV = 32
N_RINGS = 4


def _hamiltonian_cycle():
    path = []
    for y in range(4):
        zs = range(4) if y % 2 == 0 else range(3, -1, -1)
        path.extend((y, z) for z in zs)
    cyc = [(0, y, z) for (y, z) in path]
    cyc += [(1, y, z) for (y, z) in reversed(path)]

    def midx(x, y, z):
        return z * 8 + y * 2 + (x if y % 2 == 0 else 1 - x)

    ring = [midx(*c) for c in cyc]
    assert sorted(ring) == list(range(N_DEV))
    inv = [0] * N_DEV
    for p, m in enumerate(ring):
        inv[m] = p
    return ring, inv

_RING, _INV = _hamiltonian_cycle()


def _gemm_allreduce(x, w, scal):
    M = x.shape[0]
    N = w.shape[1]
    Q = M // N_RINGS
    CH = Q // N_DEV
    QCH = 64

    def body(scal_ref, x_ref, w_ref, out_ref, send_bufs, recv_bufs, qbuf,
             send_sems, recv_sems, store_sems,
             qload_sems, qstore_sems, credits_sem, amax_ref):
        pos = scal_ref[0]
        left = scal_ref[1]
        right = scal_ref[2]
        amax_ref[0] = jnp.float32(0.0)

        barrier_sem = pltpu.get_barrier_semaphore()
        for nbr in (left, right):
            pl.semaphore_signal(
                barrier_sem, inc=1,
                device_id=(nbr,), device_id_type=pl.DeviceIdType.MESH,
            )
        pl.semaphore_wait(barrier_sem, 2)

        def fwd(rr):
            return rr < 2

        def peer(rr):
            return right if fwd(rr) else left

        def crediter(rr):
            return left if fwd(rr) else right

        def row0(rr, idx):
            return rr * Q + idx * CH

        def partial(rr, idx):
            return jax.lax.dot_general(
                x_ref[pl.ds(row0(rr, idx), CH), :], w_ref[:, :],
                dimension_numbers=(((1,), (0,)), ((), ())),
                preferred_element_type=jnp.float32,
                precision=lax.Precision.HIGHEST)

        def store(rr, src_slot, idx, from_send):
            buf = send_bufs if from_send else recv_bufs
            cp = pltpu.make_async_copy(
                buf.at[rr, src_slot],
                out_ref.at[pl.ds(row0(rr, idx), CH), :],
                store_sems.at[rr])
            cp.start()
            return cp

        def rdma(rr, slot):
            return pltpu.make_async_remote_copy(
                src_ref=send_bufs.at[rr, slot],
                dst_ref=recv_bufs.at[rr, slot],
                send_sem=send_sems.at[rr, slot],
                recv_sem=recv_sems.at[rr, slot],
                device_id=(peer(rr),),
                device_id_type=pl.DeviceIdType.MESH,
            )

        def give_credits():
            for rr in range(N_RINGS):
                pl.semaphore_signal(
                    credits_sem.at[rr], inc=1,
                    device_id=(crediter(rr),),
                    device_id_type=pl.DeviceIdType.MESH,
                )

        def wait_credits(g):
            @pl.when(g >= 2)
            def _():
                for rr in range(N_RINGS):
                    pl.semaphore_wait(credits_sem.at[rr], 1)

        def rs_acc_idx(rr, g):
            if fwd(rr):
                return lax.rem(pos - (g + 1) + N_DEV, N_DEV)
            return lax.rem(pos + (g + 1), N_DEV)

        def ag_idx(rr, t):
            if fwd(rr):
                return lax.rem(pos - t + N_DEV, N_DEV)
            return lax.rem(pos + t, N_DEV)

        for rr in range(N_RINGS):
            send_bufs[rr, 0] = partial(rr, pos)
            rdma(rr, 0).start()

        def rs_tick(t, carry):
            slot = lax.rem(t, 2)
            nxt = lax.rem(t + 1, 2)
            for rr in range(N_RINGS):
                @pl.when(t >= 1)
                def _():
                    rdma(rr, nxt).wait_send()

                send_bufs[rr, nxt] = partial(rr, rs_acc_idx(rr, t))
                rdma(rr, slot).wait_recv()
                send_bufs[rr, nxt] = (send_bufs[rr, nxt]
                                      + recv_bufs[rr, slot])
                pl.semaphore_signal(
                    credits_sem.at[rr], inc=1,
                    device_id=(crediter(rr),),
                    device_id_type=pl.DeviceIdType.MESH,
                )

                @pl.when(t + 1 < N_DEV - 1)
                def _():
                    @pl.when(t >= 1)
                    def _():
                        pl.semaphore_wait(credits_sem.at[rr], 1)
                    rdma(rr, nxt).start()
            return carry

        lax.fori_loop(0, N_DEV - 1, rs_tick, 0)
        for rr in range(N_RINGS):
            rdma(rr, 0).wait_send()

        for rr in range(N_RINGS):
            pl.semaphore_wait(credits_sem.at[rr], 1)
            rdma(rr, 1).start()

        amax = jnp.float32(0.0)
        sts = []
        for rr in range(N_RINGS):
            own = lax.rem(pos + (1 if fwd(rr) else N_DEV - 1), N_DEV)
            sts.append(store(rr, 1, own, from_send=True))
            amax = jnp.maximum(amax, jnp.max(jnp.abs(send_bufs[rr, 1])))
        amax_ref[0] = amax
        for st in sts:
            st.wait()

        def ag_tick(t, carry):
            r_s = lax.rem(t + 1, 2)
            s_s = lax.rem(t, 2)
            for rr in range(N_RINGS):
                rdma(rr, r_s).wait_recv()
                store(rr, r_s, ag_idx(rr, t), from_send=False)

                @pl.when(t >= 1)
                def _():
                    rdma(rr, s_s).wait_send()

                @pl.when(t + 1 < N_DEV - 1)
                def _():
                    send_bufs[rr, s_s] = recv_bufs[rr, r_s]
                    pl.semaphore_wait(credits_sem.at[rr], 1)
                    rdma(rr, s_s).start()

                pltpu.make_async_copy(
                    recv_bufs.at[rr, r_s],
                    out_ref.at[pl.ds(row0(rr, ag_idx(rr, t)), CH), :],
                    store_sems.at[rr]).wait()
                pl.semaphore_signal(
                    credits_sem.at[rr], inc=1,
                    device_id=(crediter(rr),),
                    device_id_type=pl.DeviceIdType.MESH,
                )
            m = amax_ref[0]
            for rr in range(N_RINGS):
                m = jnp.maximum(m, jnp.max(jnp.abs(recv_bufs[rr, r_s])))
            amax_ref[0] = m
            return carry

        lax.fori_loop(0, N_DEV - 1, ag_tick, 0)

        for rr in range(N_RINGS):
            rdma(rr, 1).wait_send()
        for rr in range(N_RINGS):
            pl.semaphore_wait(credits_sem.at[rr], 2)

        amax = amax_ref[0]
        scale = amax / jnp.float32(448.0)
        inv_scale = jnp.float32(448.0) / amax

        def snap(v):
            a = jnp.abs(v) * inv_scale
            u = lax.bitcast_convert_type(a, jnp.int32)
            lsb = jnp.bitwise_and(lax.shift_right_logical(u, 20), 1)
            ur = jnp.bitwise_and(u + lsb + ((1 << 19) - 1),
                                 jnp.int32(~((1 << 20) - 1)))
            n_norm = lax.bitcast_convert_type(ur, jnp.float32)
            magic = jnp.float32(16384.0)
            n_sub = (a + magic) - magic
            snapped = jnp.where(a >= jnp.float32(2.0 ** -6), n_norm, n_sub)
            snapped = jnp.minimum(snapped, jnp.float32(448.0))
            return jnp.sign(v) * snapped * scale

        n_q = M // QCH

        def qload_cp(k):
            slot = lax.rem(k, 2)
            return pltpu.make_async_copy(
                out_ref.at[pl.ds(k * QCH, QCH), :], qbuf.at[slot],
                qload_sems.at[slot])

        def qstore_cp(k):
            slot = lax.rem(k, 2)
            return pltpu.make_async_copy(
                qbuf.at[slot], out_ref.at[pl.ds(k * QCH, QCH), :],
                qstore_sems.at[slot])

        qload_cp(0).start()

        def qbody(k, carry):
            slot = lax.rem(k, 2)

            @pl.when(k >= 1)
            def _():
                qstore_cp(k - 1).wait()

            @pl.when(k + 1 < n_q)
            def _():
                qload_cp(k + 1).start()

            qload_cp(k).wait()
            qbuf[slot] = snap(qbuf[slot])
            qstore_cp(k).start()
            return carry

        lax.fori_loop(0, n_q, qbody, 0)
        qstore_cp(n_q - 1).wait()

    return pl.pallas_call(
        body,
        out_shape=jax.ShapeDtypeStruct((M, N), jnp.float32),
        in_specs=[
            pl.BlockSpec(memory_space=pltpu.MemorySpace.SMEM),
            pl.BlockSpec(memory_space=pltpu.MemorySpace.VMEM),
            pl.BlockSpec(memory_space=pltpu.MemorySpace.VMEM),
        ],
        out_specs=pl.BlockSpec(memory_space=pl.ANY),
        scratch_shapes=[
            pltpu.VMEM((N_RINGS, 2, CH, N), jnp.float32),
            pltpu.VMEM((N_RINGS, 2, CH, N), jnp.float32),
            pltpu.VMEM((2, QCH, N), jnp.float32),
            pltpu.SemaphoreType.DMA((N_RINGS, 2)),
            pltpu.SemaphoreType.DMA((N_RINGS, 2)),
            pltpu.SemaphoreType.DMA((N_RINGS,)),
            pltpu.SemaphoreType.DMA((2,)),
            pltpu.SemaphoreType.DMA((2,)),
            pltpu.SemaphoreType.REGULAR((N_RINGS,)),
            pltpu.SMEM((1,), jnp.float32),
        ],
        compiler_params=pltpu.CompilerParams(collective_id=0),
    )(scal, x, w)


def kernel(x, w_mat):
    r = lax.axis_index("i")
    ring = jnp.asarray(_RING, jnp.int32)
    pos = jnp.asarray(_INV, jnp.int32)[r]
    right = ring[lax.rem(pos + 1, N_DEV)]
    left = ring[lax.rem(pos - 1 + N_DEV, N_DEV)]
    scal = jnp.stack([pos, left, right]).astype(jnp.int32)
    return _gemm_allreduce(x, w_mat, scal)


# device time: 1651182 ns/iter; 1.1826x vs baseline; 1.0129x over previous
import jax
import jax.numpy as jnp
from jax import lax
from jax.experimental import pallas as pl
from jax.experimental.pallas import tpu as pltpu

N_DEV = 32
N_RINGS = 4


def _hamiltonian_cycle():
    path = []
    for y in range(4):
        zs = range(4) if y % 2 == 0 else range(3, -1, -1)
        path.extend((y, z) for z in zs)
    cyc = [(0, y, z) for (y, z) in path]
    cyc += [(1, y, z) for (y, z) in reversed(path)]

    def midx(x, y, z):
        return z * 8 + y * 2 + (x if y % 2 == 0 else 1 - x)

    ring = [midx(*c) for c in cyc]
    assert sorted(ring) == list(range(N_DEV))
    inv = [0] * N_DEV
    for p, m in enumerate(ring):
        inv[m] = p
    return ring, inv

_RING, _INV = _hamiltonian_cycle()


def _gemm_allreduce(x, w, scal):
    M = x.shape[0]
    N = w.shape[1]
    Q = M // N_RINGS
    CH = Q // N_DEV
    QCH = 128

    def body(scal_ref, x_ref, w_ref, out_ref, send_bufs, recv_bufs, qbuf,
             send_sems, recv_sems, store_sems,
             qload_sems, qstore_sems, credits_sem, amax_ref):
        pos = scal_ref[0]
        left = scal_ref[1]
        right = scal_ref[2]
        amax_ref[0] = jnp.float32(0.0)

        barrier_sem = pltpu.get_barrier_semaphore()
        for nbr in (left, right):
            pl.semaphore_signal(
                barrier_sem, inc=1,
                device_id=(nbr,), device_id_type=pl.DeviceIdType.MESH,
            )
        pl.semaphore_wait(barrier_sem, 2)

        def fwd(rr):
            return rr < 2

        def peer(rr):
            return right if fwd(rr) else left

        def crediter(rr):
            return left if fwd(rr) else right

        def row0(rr, idx):
            return rr * Q + idx * CH

        def partial(rr, idx):
            return jax.lax.dot_general(
                x_ref[pl.ds(row0(rr, idx), CH), :], w_ref[:, :],
                dimension_numbers=(((1,), (0,)), ((), ())),
                preferred_element_type=jnp.float32,
                precision=lax.Precision.HIGHEST)

        def store(rr, src_slot, idx, from_send):
            buf = send_bufs if from_send else recv_bufs
            cp = pltpu.make_async_copy(
                buf.at[rr, src_slot],
                out_ref.at[pl.ds(row0(rr, idx), CH), :],
                store_sems.at[rr])
            cp.start()
            return cp

        def rdma(rr, slot):
            return pltpu.make_async_remote_copy(
                src_ref=send_bufs.at[rr, slot],
                dst_ref=recv_bufs.at[rr, slot],
                send_sem=send_sems.at[rr, slot],
                recv_sem=recv_sems.at[rr, slot],
                device_id=(peer(rr),),
                device_id_type=pl.DeviceIdType.MESH,
            )

        def give_credits():
            for rr in range(N_RINGS):
                pl.semaphore_signal(
                    credits_sem.at[rr], inc=1,
                    device_id=(crediter(rr),),
                    device_id_type=pl.DeviceIdType.MESH,
                )

        def wait_credits(g):
            @pl.when(g >= 2)
            def _():
                for rr in range(N_RINGS):
                    pl.semaphore_wait(credits_sem.at[rr], 1)

        def rs_acc_idx(rr, g):
            if fwd(rr):
                return lax.rem(pos - (g + 1) + N_DEV, N_DEV)
            return lax.rem(pos + (g + 1), N_DEV)

        def ag_idx(rr, t):
            if fwd(rr):
                return lax.rem(pos - t + N_DEV, N_DEV)
            return lax.rem(pos + t, N_DEV)

        for rr in range(N_RINGS):
            send_bufs[rr, 0] = partial(rr, pos)
            rdma(rr, 0).start()

        def rs_tick(t, carry):
            slot = lax.rem(t, 2)
            nxt = lax.rem(t + 1, 2)
            for rr in range(N_RINGS):
                @pl.when(t >= 1)
                def _():
                    rdma(rr, nxt).wait_send()

                send_bufs[rr, nxt] = partial(rr, rs_acc_idx(rr, t))
                rdma(rr, slot).wait_recv()
                send_bufs[rr, nxt] = (send_bufs[rr, nxt]
                                      + recv_bufs[rr, slot])
                pl.semaphore_signal(
                    credits_sem.at[rr], inc=1,
                    device_id=(crediter(rr),),
                    device_id_type=pl.DeviceIdType.MESH,
                )

                @pl.when(t + 1 < N_DEV - 1)
                def _():
                    @pl.when(t >= 1)
                    def _():
                        pl.semaphore_wait(credits_sem.at[rr], 1)
                    rdma(rr, nxt).start()
            return carry

        lax.fori_loop(0, N_DEV - 1, rs_tick, 0)
        for rr in range(N_RINGS):
            rdma(rr, 0).wait_send()

        for rr in range(N_RINGS):
            pl.semaphore_wait(credits_sem.at[rr], 1)
            rdma(rr, 1).start()

        amax = jnp.float32(0.0)
        sts = []
        for rr in range(N_RINGS):
            own = lax.rem(pos + (1 if fwd(rr) else N_DEV - 1), N_DEV)
            sts.append(store(rr, 1, own, from_send=True))
            amax = jnp.maximum(amax, jnp.max(jnp.abs(send_bufs[rr, 1])))
        amax_ref[0] = amax
        for st in sts:
            st.wait()

        def ag_tick(t, carry):
            r_s = lax.rem(t + 1, 2)
            s_s = lax.rem(t, 2)
            for rr in range(N_RINGS):
                rdma(rr, r_s).wait_recv()
                store(rr, r_s, ag_idx(rr, t), from_send=False)

                @pl.when(t >= 1)
                def _():
                    rdma(rr, s_s).wait_send()

                @pl.when(t + 1 < N_DEV - 1)
                def _():
                    send_bufs[rr, s_s] = recv_bufs[rr, r_s]
                    pl.semaphore_wait(credits_sem.at[rr], 1)
                    rdma(rr, s_s).start()

                pltpu.make_async_copy(
                    recv_bufs.at[rr, r_s],
                    out_ref.at[pl.ds(row0(rr, ag_idx(rr, t)), CH), :],
                    store_sems.at[rr]).wait()
                pl.semaphore_signal(
                    credits_sem.at[rr], inc=1,
                    device_id=(crediter(rr),),
                    device_id_type=pl.DeviceIdType.MESH,
                )
            m = amax_ref[0]
            for rr in range(N_RINGS):
                m = jnp.maximum(m, jnp.max(jnp.abs(recv_bufs[rr, r_s])))
            amax_ref[0] = m
            return carry

        lax.fori_loop(0, N_DEV - 1, ag_tick, 0)

        for rr in range(N_RINGS):
            rdma(rr, 1).wait_send()
        for rr in range(N_RINGS):
            pl.semaphore_wait(credits_sem.at[rr], 2)

        amax = amax_ref[0]
        scale = amax / jnp.float32(448.0)
        inv_scale = jnp.float32(448.0) / amax

        def snap(v):
            a = jnp.abs(v) * inv_scale
            u = lax.bitcast_convert_type(a, jnp.int32)
            lsb = jnp.bitwise_and(lax.shift_right_logical(u, 20), 1)
            ur = jnp.bitwise_and(u + lsb + ((1 << 19) - 1),
                                 jnp.int32(~((1 << 20) - 1)))
            n_norm = lax.bitcast_convert_type(ur, jnp.float32)
            magic = jnp.float32(16384.0)
            n_sub = (a + magic) - magic
            snapped = jnp.where(a >= jnp.float32(2.0 ** -6), n_norm, n_sub)
            snapped = jnp.minimum(snapped, jnp.float32(448.0))
            return jnp.sign(v) * snapped * scale

        n_q = M // QCH

        def qload_cp(k):
            slot = lax.rem(k, 2)
            return pltpu.make_async_copy(
                out_ref.at[pl.ds(k * QCH, QCH), :], qbuf.at[slot],
                qload_sems.at[slot])

        def qstore_cp(k):
            slot = lax.rem(k, 2)
            return pltpu.make_async_copy(
                qbuf.at[slot], out_ref.at[pl.ds(k * QCH, QCH), :],
                qstore_sems.at[slot])

        qload_cp(0).start()

        def qbody(k, carry):
            slot = lax.rem(k, 2)

            @pl.when(k >= 1)
            def _():
                qstore_cp(k - 1).wait()

            @pl.when(k + 1 < n_q)
            def _():
                qload_cp(k + 1).start()

            qload_cp(k).wait()
            qbuf[slot] = snap(qbuf[slot])
            qstore_cp(k).start()
            return carry

        lax.fori_loop(0, n_q, qbody, 0)
        qstore_cp(n_q - 1).wait()

    return pl.pallas_call(
        body,
        out_shape=jax.ShapeDtypeStruct((M, N), jnp.float32),
        in_specs=[
            pl.BlockSpec(memory_space=pltpu.MemorySpace.SMEM),
            pl.BlockSpec(memory_space=pltpu.MemorySpace.VMEM),
            pl.BlockSpec(memory_space=pltpu.MemorySpace.VMEM),
        ],
        out_specs=pl.BlockSpec(memory_space=pl.ANY),
        scratch_shapes=[
            pltpu.VMEM((N_RINGS, 2, CH, N), jnp.float32),
            pltpu.VMEM((N_RINGS, 2, CH, N), jnp.float32),
            pltpu.VMEM((2, QCH, N), jnp.float32),
            pltpu.SemaphoreType.DMA((N_RINGS, 2)),
            pltpu.SemaphoreType.DMA((N_RINGS, 2)),
            pltpu.SemaphoreType.DMA((N_RINGS,)),
            pltpu.SemaphoreType.DMA((2,)),
            pltpu.SemaphoreType.DMA((2,)),
            pltpu.SemaphoreType.REGULAR((N_RINGS,)),
            pltpu.SMEM((1,), jnp.float32),
        ],
        compiler_params=pltpu.CompilerParams(collective_id=0),
    )(scal, x, w)


def kernel(x, w_mat):
    r = lax.axis_index("i")
    ring = jnp.asarray(_RING, jnp.int32)
    pos = jnp.asarray(_INV, jnp.int32)[r]
    right = ring[lax.rem(pos + 1, N_DEV)]
    left = ring[lax.rem(pos - 1 + N_DEV, N_DEV)]
    scal = jnp.stack([pos, left, right]).astype(jnp.int32)
    return _gemm_allreduce(x, w_mat, scal)


# device time: 1651124 ns/iter; 1.1826x vs baseline; 1.0000x over previous
import jax
import jax.numpy as jnp
from jax import lax
from jax.experimental import pallas as pl
from jax.experimental.pallas import tpu as pltpu

N_DEV = 32
N_RINGS = 4


def _hamiltonian_cycle():
    path = []
    for y in range(4):
        zs = range(4) if y % 2 == 0 else range(3, -1, -1)
        path.extend((y, z) for z in zs)
    cyc = [(0, y, z) for (y, z) in path]
    cyc += [(1, y, z) for (y, z) in reversed(path)]

    def midx(x, y, z):
        return z * 8 + y * 2 + (x if y % 2 == 0 else 1 - x)

    ring = [midx(*c) for c in cyc]
    assert sorted(ring) == list(range(N_DEV))
    inv = [0] * N_DEV
    for p, m in enumerate(ring):
        inv[m] = p
    return ring, inv

_RING, _INV = _hamiltonian_cycle()


def _gemm_allreduce(x, w, scal):
    M = x.shape[0]
    N = w.shape[1]
    Q = M // N_RINGS
    CH = Q // N_DEV
    QCH = 128

    def body(scal_ref, x_ref, w_ref, out_ref, send_bufs, recv_bufs, qbuf,
             send_sems, recv_sems, store_sems,
             qload_sems, qstore_sems, credits_sem, amax_ref):
        pos = scal_ref[0]
        left = scal_ref[1]
        right = scal_ref[2]
        amax_ref[0] = jnp.float32(0.0)

        barrier_sem = pltpu.get_barrier_semaphore()
        for nbr in (left, right):
            pl.semaphore_signal(
                barrier_sem, inc=1,
                device_id=(nbr,), device_id_type=pl.DeviceIdType.MESH,
            )
        pl.semaphore_wait(barrier_sem, 2)

        def fwd(rr):
            return rr < 2

        def peer(rr):
            return right if fwd(rr) else left

        def crediter(rr):
            return left if fwd(rr) else right

        def row0(rr, idx):
            return rr * Q + idx * CH

        def partial(rr, idx):
            return jax.lax.dot_general(
                x_ref[pl.ds(row0(rr, idx), CH), :], w_ref[:, :],
                dimension_numbers=(((1,), (0,)), ((), ())),
                preferred_element_type=jnp.float32,
                precision=lax.Precision.HIGHEST)

        def store(rr, src_slot, idx, from_send):
            buf = send_bufs if from_send else recv_bufs
            cp = pltpu.make_async_copy(
                buf.at[rr, src_slot],
                out_ref.at[pl.ds(row0(rr, idx), CH), :],
                store_sems.at[rr])
            cp.start()
            return cp

        def rdma(rr, slot):
            return pltpu.make_async_remote_copy(
                src_ref=send_bufs.at[rr, slot],
                dst_ref=recv_bufs.at[rr, slot],
                send_sem=send_sems.at[rr, slot],
                recv_sem=recv_sems.at[rr, slot],
                device_id=(peer(rr),),
                device_id_type=pl.DeviceIdType.MESH,
            )

        def rs_acc_idx(rr, g):
            if fwd(rr):
                return lax.rem(pos - (g + 1) + N_DEV, N_DEV)
            return lax.rem(pos + (g + 1), N_DEV)

        def ag_idx(rr, t):
            if fwd(rr):
                return lax.rem(pos - t + N_DEV, N_DEV)
            return lax.rem(pos + t, N_DEV)

        for rr in range(N_RINGS):
            send_bufs[rr, 0] = partial(rr, pos)
            rdma(rr, 0).start()

        def rs_tick(t, carry):
            slot = lax.rem(t, 2)
            nxt = lax.rem(t + 1, 2)
            for rr in range(N_RINGS):
                @pl.when(t >= 1)
                def _():
                    rdma(rr, nxt).wait_send()

                send_bufs[rr, nxt] = partial(rr, rs_acc_idx(rr, t))
                rdma(rr, slot).wait_recv()
                send_bufs[rr, nxt] = (send_bufs[rr, nxt]
                                      + recv_bufs[rr, slot])
                pl.semaphore_signal(
                    credits_sem.at[rr], inc=1,
                    device_id=(crediter(rr),),
                    device_id_type=pl.DeviceIdType.MESH,
                )

                @pl.when(t + 1 < N_DEV - 1)
                def _():
                    @pl.when(t >= 1)
                    def _():
                        pl.semaphore_wait(credits_sem.at[rr], 1)
                    rdma(rr, nxt).start()
            return carry

        lax.fori_loop(0, N_DEV - 1, rs_tick, 0)
        for rr in range(N_RINGS):
            rdma(rr, 0).wait_send()

        for rr in range(N_RINGS):
            pl.semaphore_wait(credits_sem.at[rr], 1)
            rdma(rr, 1).start()

        amax = jnp.float32(0.0)
        sts = []
        for rr in range(N_RINGS):
            own = lax.rem(pos + (1 if fwd(rr) else N_DEV - 1), N_DEV)
            sts.append(store(rr, 1, own, from_send=True))
            amax = jnp.maximum(amax, jnp.max(jnp.abs(send_bufs[rr, 1])))
        amax_ref[0] = amax
        for st in sts:
            st.wait()

        def ag_tick(t, carry):
            r_s = lax.rem(t + 1, 2)
            s_s = lax.rem(t, 2)
            for rr in range(N_RINGS):
                rdma(rr, r_s).wait_recv()
                store(rr, r_s, ag_idx(rr, t), from_send=False)

                @pl.when(t >= 1)
                def _():
                    rdma(rr, s_s).wait_send()

                @pl.when(t + 1 < N_DEV - 1)
                def _():
                    send_bufs[rr, s_s] = recv_bufs[rr, r_s]
                    pl.semaphore_wait(credits_sem.at[rr], 1)
                    rdma(rr, s_s).start()

                pltpu.make_async_copy(
                    recv_bufs.at[rr, r_s],
                    out_ref.at[pl.ds(row0(rr, ag_idx(rr, t)), CH), :],
                    store_sems.at[rr]).wait()
                pl.semaphore_signal(
                    credits_sem.at[rr], inc=1,
                    device_id=(crediter(rr),),
                    device_id_type=pl.DeviceIdType.MESH,
                )
            m = amax_ref[0]
            for rr in range(N_RINGS):
                m = jnp.maximum(m, jnp.max(jnp.abs(recv_bufs[rr, r_s])))
            amax_ref[0] = m
            return carry

        lax.fori_loop(0, N_DEV - 1, ag_tick, 0)

        for rr in range(N_RINGS):
            rdma(rr, 1).wait_send()
        for rr in range(N_RINGS):
            pl.semaphore_wait(credits_sem.at[rr], 2)

        amax = amax_ref[0]
        scale = amax / jnp.float32(448.0)
        inv_scale = jnp.float32(448.0) / amax

        def snap(v):
            a = jnp.abs(v) * inv_scale
            u = lax.bitcast_convert_type(a, jnp.int32)
            lsb = jnp.bitwise_and(lax.shift_right_logical(u, 20), 1)
            ur = jnp.bitwise_and(u + lsb + ((1 << 19) - 1),
                                 jnp.int32(~((1 << 20) - 1)))
            n_norm = lax.bitcast_convert_type(ur, jnp.float32)
            magic = jnp.float32(16384.0)
            n_sub = (a + magic) - magic
            snapped = jnp.where(a >= jnp.float32(2.0 ** -6), n_norm, n_sub)
            snapped = jnp.minimum(snapped, jnp.float32(448.0))
            return jnp.sign(v) * snapped * scale

        n_q = M // QCH

        def qload_cp(k):
            slot = lax.rem(k, 2)
            return pltpu.make_async_copy(
                out_ref.at[pl.ds(k * QCH, QCH), :], qbuf.at[slot],
                qload_sems.at[slot])

        def qstore_cp(k):
            slot = lax.rem(k, 2)
            return pltpu.make_async_copy(
                qbuf.at[slot], out_ref.at[pl.ds(k * QCH, QCH), :],
                qstore_sems.at[slot])

        qload_cp(0).start()

        def qbody(k, carry):
            slot = lax.rem(k, 2)

            @pl.when(k >= 1)
            def _():
                qstore_cp(k - 1).wait()

            @pl.when(k + 1 < n_q)
            def _():
                qload_cp(k + 1).start()

            qload_cp(k).wait()
            qbuf[slot] = snap(qbuf[slot])
            qstore_cp(k).start()
            return carry

        lax.fori_loop(0, n_q, qbody, 0)
        qstore_cp(n_q - 1).wait()

    return pl.pallas_call(
        body,
        out_shape=jax.ShapeDtypeStruct((M, N), jnp.float32),
        in_specs=[
            pl.BlockSpec(memory_space=pltpu.MemorySpace.SMEM),
            pl.BlockSpec(memory_space=pltpu.MemorySpace.VMEM),
            pl.BlockSpec(memory_space=pltpu.MemorySpace.VMEM),
        ],
        out_specs=pl.BlockSpec(memory_space=pl.ANY),
        scratch_shapes=[
            pltpu.VMEM((N_RINGS, 2, CH, N), jnp.float32),
            pltpu.VMEM((N_RINGS, 2, CH, N), jnp.float32),
            pltpu.VMEM((2, QCH, N), jnp.float32),
            pltpu.SemaphoreType.DMA((N_RINGS, 2)),
            pltpu.SemaphoreType.DMA((N_RINGS, 2)),
            pltpu.SemaphoreType.DMA((N_RINGS,)),
            pltpu.SemaphoreType.DMA((2,)),
            pltpu.SemaphoreType.DMA((2,)),
            pltpu.SemaphoreType.REGULAR((N_RINGS,)),
            pltpu.SMEM((1,), jnp.float32),
        ],
        compiler_params=pltpu.CompilerParams(collective_id=0),
    )(scal, x, w)


def kernel(x, w_mat):
    r = lax.axis_index("i")
    ring = jnp.asarray(_RING, jnp.int32)
    pos = jnp.asarray(_INV, jnp.int32)[r]
    right = ring[lax.rem(pos + 1, N_DEV)]
    left = ring[lax.rem(pos - 1 + N_DEV, N_DEV)]
    scal = jnp.stack([pos, left, right]).astype(jnp.int32)
    return _gemm_allreduce(x, w_mat, scal)
